# baseline (speedup 1.0000x reference)
import functools

import jax
import jax.numpy as jnp
from jax import lax
from jax.experimental import pallas as pl
from jax.experimental.pallas import tpu as pltpu
from jax.experimental.pallas import tpu_sc as plsc

B, T = 1024, 200
N = B * T
DIMS = (32, 16, 16, 16)
OFFS = (0, 32, 48, 64)
PAD = 128
OUT_DIM = 128

NC, NS = 2, 16
NW = NC * NS
ROWS_PER_W = N // NW
IDX_LANES = 128
IDX_ROWS_PER_W = ROWS_PER_W // IDX_LANES
CHUNK_IDX_ROWS = 5
CHUNK = CHUNK_IDX_ROWS * IDX_LANES
NCHUNK = IDX_ROWS_PER_W // CHUNK_IDX_ROWS


V0 = 1000000
TCH = 800
TUNITS = V0 // TCH
TGROUPS = TCH // 16


def _transpose_body(et, out, ta, tb, ob, sem_a, sem_b):
    wid = lax.axis_index("s") * NC + lax.axis_index("c")

    def unit_id(k):
        return wid + NW * k

    def start_in(k, buf, sem):
        u = unit_id(k)

        @pl.when(u < TUNITS)
        def _():
            pltpu.make_async_copy(
                et.at[:, pl.ds(u * TCH, TCH)], buf, sem).start()

    def do_unit(k, buf, sem):
        u = unit_id(k)

        @pl.when(u < TUNITS)
        def _():
            pltpu.make_async_copy(
                et.at[:, pl.ds(u * TCH, TCH)], buf, sem).wait()

            def grp(g, carry):
                rows = g * 16 + lax.iota(jnp.int32, 16)
                for j in range(32):
                    v = buf[j, pl.ds(g * 16, 16)]
                    plsc.store_scatter(ob, [rows, jnp.full((16,), j, jnp.int32)], v)
                return carry

            lax.fori_loop(0, TGROUPS, grp, 0)
            pltpu.sync_copy(ob, out.at[pl.ds(u * TCH, TCH)])

    start_in(0, ta, sem_a)

    def pair(k2, carry):
        k = 2 * k2
        start_in(k + 1, tb, sem_b)
        do_unit(k, ta, sem_a)
        start_in(k + 2, ta, sem_a)
        do_unit(k + 1, tb, sem_b)
        return carry

    lax.fori_loop(0, 20, pair, 0)


@jax.jit
def _sc_transpose(et):
    mesh = plsc.VectorSubcoreMesh(core_axis_name="c", subcore_axis_name="s")
    return pl.kernel(
        _transpose_body,
        out_type=jax.ShapeDtypeStruct((V0, 32), jnp.float32),
        mesh=mesh,
        scratch_types=[
            pltpu.VMEM((32, TCH), jnp.float32),
            pltpu.VMEM((32, TCH), jnp.float32),
            pltpu.VMEM((TCH, 32), jnp.float32),
            pltpu.SemaphoreType.DMA,
            pltpu.SemaphoreType.DMA,
        ],
        compiler_params=pltpu.CompilerParams(
            use_tc_tiling_on_sc=False, needs_layout_passes=False),
    )(et)


def _gather_body(f0, f1, f2, f3, e0, e1, e2, e3, out,
                 i0, i1, i2, i3, r0, r1, r2, r3, zbuf, sem):
    wid = lax.axis_index("s") * NC + lax.axis_index("c")
    base_r = wid * ROWS_PER_W

    pltpu.sync_copy(f0.at[wid], i0)
    pltpu.sync_copy(f1.at[wid], i1)
    pltpu.sync_copy(f2.at[wid], i2)
    pltpu.sync_copy(f3.at[wid], i3)

    zero = jnp.zeros((16,), jnp.float32)

    def zrow(r, carry):
        zbuf[r, pl.ds(0, 16)] = zero
        zbuf[r, pl.ds(16, 16)] = zero
        zbuf[r, pl.ds(32, 16)] = zero
        return carry

    lax.fori_loop(0, CHUNK, zrow, 0)

    tabs = (e0, e1, e2, e3)
    idxs = (i0, i1, i2, i3)
    rbufs = (r0, r1, r2, r3)

    def chunk(c, carry):
        cps = []
        for t in range(4):
            for j in range(CHUNK_IDX_ROWS):
                cps.append(pltpu.make_async_copy(
                    tabs[t].at[idxs[t].at[c * CHUNK_IDX_ROWS + j]],
                    rbufs[t].at[pl.ds(j * IDX_LANES, IDX_LANES)],
                    sem,
                ))
        for cp in cps:
            cp.start()
        for cp in cps:
            cp.wait()
        rows = pl.ds(base_r + c * CHUNK, CHUNK)
        for t in range(4):
            pltpu.sync_copy(rbufs[t], out.at[rows, pl.ds(OFFS[t], DIMS[t])])
        pltpu.sync_copy(zbuf, out.at[rows, pl.ds(80, 48)])
        return carry

    lax.fori_loop(0, NCHUNK, chunk, 0)


@jax.jit
def _sc_gather(f0, f1, f2, f3, e0, e1, e2, e3):
    mesh = plsc.VectorSubcoreMesh(core_axis_name="c", subcore_axis_name="s")
    return pl.kernel(
        _gather_body,
        out_type=jax.ShapeDtypeStruct((N, PAD), jnp.float32),
        mesh=mesh,
        scratch_types=[
            pltpu.VMEM((IDX_ROWS_PER_W, IDX_LANES), jnp.int32),
            pltpu.VMEM((IDX_ROWS_PER_W, IDX_LANES), jnp.int32),
            pltpu.VMEM((IDX_ROWS_PER_W, IDX_LANES), jnp.int32),
            pltpu.VMEM((IDX_ROWS_PER_W, IDX_LANES), jnp.int32),
            pltpu.VMEM((CHUNK, 32), jnp.float32),
            pltpu.VMEM((CHUNK, 16), jnp.float32),
            pltpu.VMEM((CHUNK, 16), jnp.float32),
            pltpu.VMEM((CHUNK, 16), jnp.float32),
            pltpu.VMEM((CHUNK, 48), jnp.float32),
            pltpu.SemaphoreType.DMA,
        ],
        compiler_params=pltpu.CompilerParams(use_tc_tiling_on_sc=False),
    )(f0, f1, f2, f3, e0, e1, e2, e3)


MM_BLK = 2048


def _mm_body(s, w, bias, o):
    o[...] = jnp.dot(s[...], w[...],
                     preferred_element_type=jnp.float32) + bias[0:1, :]


@jax.jit
def _tc_project(s, w, bias):
    return pl.pallas_call(
        _mm_body,
        grid=(N // MM_BLK,),
        in_specs=[
            pl.BlockSpec((MM_BLK, PAD), lambda i: (i, 0)),
            pl.BlockSpec((PAD, OUT_DIM), lambda i: (0, 0)),
            pl.BlockSpec((8, OUT_DIM), lambda i: (0, 0)),
        ],
        out_specs=pl.BlockSpec((MM_BLK, OUT_DIM), lambda i: (i, 0)),
        out_shape=jax.ShapeDtypeStruct((N, OUT_DIM), jnp.float32),
    )(s, w, bias)


def kernel(feat0, feat1, feat2, feat3, E0, E1, E2, E3, W, b):
    fs = [f.reshape(NW, IDX_ROWS_PER_W, IDX_LANES)
          for f in (feat0, feat1, feat2, feat3)]
    E0R = _sc_transpose(jnp.transpose(E0))
    s = _sc_gather(*fs, E0R, E1, E2, E3)
    wp = jnp.zeros((PAD, OUT_DIM), jnp.float32).at[0:80, :].set(W)
    bias = jnp.broadcast_to(b, (8, OUT_DIM))
    out = _tc_project(s, wp, bias)
    return out.reshape(B, T, OUT_DIM)


# speedup vs baseline: 3.7370x; 3.7370x over previous
import functools

import jax
import jax.numpy as jnp
from jax import lax
from jax.experimental import pallas as pl
from jax.experimental.pallas import tpu as pltpu
from jax.experimental.pallas import tpu_sc as plsc

B, T = 1024, 200
N = B * T
DIMS = (32, 16, 16, 16)
OFFS = (0, 32, 48, 64)
PAD = 128
OUT_DIM = 128

NC, NS = 2, 16
NW = NC * NS
ROWS_PER_W = N // NW
IDX_LANES = 128
IDX_ROWS_PER_W = ROWS_PER_W // IDX_LANES
CHUNK_IDX_ROWS = 5
CHUNK = CHUNK_IDX_ROWS * IDX_LANES
NCHUNK = IDX_ROWS_PER_W // CHUNK_IDX_ROWS

V0 = 1000000
PBN = 2048
PROWS = PBN // 4
NPACK = (V0 + PBN - 1) // PBN
P0_ROWS = NPACK * PROWS


def _pack_body(x, o):
    acc = None
    for g in range(4):
        y = lax.dot_general(
            x[:, PROWS * g:PROWS * (g + 1)],
            jnp.eye(32, 128, 32 * g, dtype=jnp.float32),
            (((0,), (0,)), ((), ())),
            preferred_element_type=jnp.float32)
        acc = y if acc is None else acc + y
    o[...] = acc


@jax.jit
def _tc_pack(e0t):
    return pl.pallas_call(
        _pack_body,
        grid=(NPACK,),
        in_specs=[pl.BlockSpec((32, PBN), lambda i: (0, i))],
        out_specs=pl.BlockSpec((PROWS, PAD), lambda i: (i, 0)),
        out_shape=jax.ShapeDtypeStruct((P0_ROWS, PAD), jnp.float32),
    )(e0t)


def _gather_body(g0i, f1, f2, f3, q0, p0, e1, e2, e3, out,
                 i0, i1, i2, i3, qb, g0, r0, r1, r2, r3, zbuf,
                 sem, sga, sgb):
    wid = lax.axis_index("s") * NC + lax.axis_index("c")
    base_r = wid * ROWS_PER_W

    pltpu.sync_copy(g0i.at[wid], i0)
    pltpu.sync_copy(f1.at[wid], i1)
    pltpu.sync_copy(f2.at[wid], i2)
    pltpu.sync_copy(f3.at[wid], i3)
    pltpu.sync_copy(q0.at[wid], qb)

    zero = jnp.zeros((16,), jnp.float32)

    def zrow(r, carry):
        zbuf[r, pl.ds(0, 16)] = zero
        zbuf[r, pl.ds(16, 16)] = zero
        zbuf[r, pl.ds(32, 16)] = zero
        return carry

    lax.fori_loop(0, IDX_LANES, zrow, 0)

    tabs = (e1, e2, e3)
    idxs = (i1, i2, i3)
    rbufs = (r1, r2, r3)
    iota16 = lax.iota(jnp.int32, 16)

    def e0_copy(c, j):
        half = j % 2
        return pltpu.make_async_copy(
            p0.at[i0.at[c * CHUNK_IDX_ROWS + j]],
            g0.at[pl.ds(half * IDX_LANES, IDX_LANES)],
            sga if half == 0 else sgb,
        )

    def chunk(c, carry):
        cps = []
        for t in range(3):
            for j in range(CHUNK_IDX_ROWS):
                cps.append(pltpu.make_async_copy(
                    tabs[t].at[idxs[t].at[c * CHUNK_IDX_ROWS + j]],
                    rbufs[t].at[pl.ds(j * IDX_LANES, IDX_LANES)],
                    sem,
                ))
        for cp in cps:
            cp.start()

        e0_copy(c, 0).start()
        for j in range(CHUNK_IDX_ROWS):
            if j + 1 < CHUNK_IDX_ROWS:
                e0_copy(c, j + 1).start()
            e0_copy(c, j).wait()
            half_off = (j % 2) * IDX_LANES

            def grp(g, carry2):
                qv = qb[c * CHUNK_IDX_ROWS + j, pl.ds(g * 16, 16)]
                col0 = qv * 32
                rowv = g * 16 + iota16 + half_off
                srow = j * IDX_LANES + g * 16 + iota16
                for jj in range(32):
                    v = plsc.load_gather(g0, [rowv, col0 + jj])
                    plsc.store_scatter(
                        r0, [srow, jnp.full((16,), jj, jnp.int32)], v)
                return carry2

            lax.fori_loop(0, IDX_LANES // 16, grp, 0)

        for cp in cps:
            cp.wait()
        rows = pl.ds(base_r + c * CHUNK, CHUNK)
        pltpu.sync_copy(r0, out.at[rows, pl.ds(0, 32)])
        for t in range(3):
            pltpu.sync_copy(rbufs[t],
                            out.at[rows, pl.ds(OFFS[t + 1], DIMS[t + 1])])
        for j in range(CHUNK_IDX_ROWS):
            pltpu.sync_copy(
                zbuf,
                out.at[pl.ds(base_r + c * CHUNK + j * IDX_LANES, IDX_LANES),
                       pl.ds(80, 48)])
        return carry

    lax.fori_loop(0, NCHUNK, chunk, 0)


@jax.jit
def _sc_gather(g0i, f1, f2, f3, q0, p0, e1, e2, e3):
    mesh = plsc.VectorSubcoreMesh(core_axis_name="c", subcore_axis_name="s")
    return pl.kernel(
        _gather_body,
        out_type=jax.ShapeDtypeStruct((N, PAD), jnp.float32),
        mesh=mesh,
        scratch_types=[
            pltpu.VMEM((IDX_ROWS_PER_W, IDX_LANES), jnp.int32),
            pltpu.VMEM((IDX_ROWS_PER_W, IDX_LANES), jnp.int32),
            pltpu.VMEM((IDX_ROWS_PER_W, IDX_LANES), jnp.int32),
            pltpu.VMEM((IDX_ROWS_PER_W, IDX_LANES), jnp.int32),
            pltpu.VMEM((IDX_ROWS_PER_W, IDX_LANES), jnp.int32),
            pltpu.VMEM((2 * IDX_LANES, PAD), jnp.float32),
            pltpu.VMEM((CHUNK, 32), jnp.float32),
            pltpu.VMEM((CHUNK, 16), jnp.float32),
            pltpu.VMEM((CHUNK, 16), jnp.float32),
            pltpu.VMEM((CHUNK, 16), jnp.float32),
            pltpu.VMEM((IDX_LANES, 48), jnp.float32),
            pltpu.SemaphoreType.DMA,
            pltpu.SemaphoreType.DMA,
            pltpu.SemaphoreType.DMA,
        ],
        compiler_params=pltpu.CompilerParams(
            use_tc_tiling_on_sc=False, needs_layout_passes=False),
    )(g0i, f1, f2, f3, q0, p0, e1, e2, e3)


MM_BLK = 2048


def _mm_body(s, w, bias, o):
    o[...] = jnp.dot(s[...], w[...],
                     preferred_element_type=jnp.float32) + bias[0:1, :]


@jax.jit
def _tc_project(s, w, bias):
    return pl.pallas_call(
        _mm_body,
        grid=(N // MM_BLK,),
        in_specs=[
            pl.BlockSpec((MM_BLK, PAD), lambda i: (i, 0)),
            pl.BlockSpec((PAD, OUT_DIM), lambda i: (0, 0)),
            pl.BlockSpec((8, OUT_DIM), lambda i: (0, 0)),
        ],
        out_specs=pl.BlockSpec((MM_BLK, OUT_DIM), lambda i: (i, 0)),
        out_shape=jax.ShapeDtypeStruct((N, OUT_DIM), jnp.float32),
    )(s, w, bias)


def kernel(feat0, feat1, feat2, feat3, E0, E1, E2, E3, W, b):
    shaped = lambda f: f.reshape(NW, IDX_ROWS_PER_W, IDX_LANES)
    g0i = shaped((feat0 >> 11) * 512 + (feat0 & 511))
    q0 = shaped((feat0 >> 9) & 3)
    fs = [shaped(f) for f in (feat1, feat2, feat3)]
    P0 = _tc_pack(jnp.transpose(E0))
    s = _sc_gather(g0i, *fs, q0, P0, E1, E2, E3)
    wp = jnp.zeros((PAD, OUT_DIM), jnp.float32).at[0:80, :].set(W)
    bias = jnp.broadcast_to(b, (8, OUT_DIM))
    out = _tc_project(s, wp, bias)
    return out.reshape(B, T, OUT_DIM)


# speedup vs baseline: 6.1972x; 1.6583x over previous
import functools

import jax
import jax.numpy as jnp
from jax import lax
from jax.experimental import pallas as pl
from jax.experimental.pallas import tpu as pltpu
from jax.experimental.pallas import tpu_sc as plsc

B, T = 1024, 200
N = B * T
DIMS = (32, 16, 16, 16)
OFFS = (0, 32, 48, 64)
PAD = 128
OUT_DIM = 128

NC, NS = 2, 16
NW = NC * NS
ROWS_PER_W = N // NW
IDX_LANES = 128
IDX_ROWS_PER_W = ROWS_PER_W // IDX_LANES
CHUNK_IDX_ROWS = 5
CHUNK = CHUNK_IDX_ROWS * IDX_LANES
NCHUNK = IDX_ROWS_PER_W // CHUNK_IDX_ROWS

V0 = 1000000
PBN = 8192
PROWS = PBN // 4
NPACK = (V0 + PBN - 1) // PBN
P0_ROWS = NPACK * PROWS
PSHIFT = 13
QSHIFT = 11


def _pack_body(x, o):
    acc = None
    for g in range(4):
        y = lax.dot_general(
            x[:, PROWS * g:PROWS * (g + 1)],
            jnp.eye(32, 128, 32 * g, dtype=jnp.float32),
            (((0,), (0,)), ((), ())),
            preferred_element_type=jnp.float32)
        acc = y if acc is None else acc + y
    o[...] = acc


@jax.jit
def _tc_pack(e0t):
    return pl.pallas_call(
        _pack_body,
        grid=(NPACK,),
        in_specs=[pl.BlockSpec((32, PBN), lambda i: (0, i))],
        out_specs=pl.BlockSpec((PROWS, PAD), lambda i: (i, 0)),
        out_shape=jax.ShapeDtypeStruct((P0_ROWS, PAD), jnp.float32),
    )(e0t)


def _gather_body(g0i, f1, f2, f3, q0, p0, e1, e2, e3, out,
                 i0, i1, i2, i3, qb, g0, r0, r1, r2, r3, zbuf,
                 sem, sga, sgb):
    wid = lax.axis_index("s") * NC + lax.axis_index("c")
    base_r = wid * ROWS_PER_W

    pltpu.sync_copy(g0i.at[wid], i0)
    pltpu.sync_copy(f1.at[wid], i1)
    pltpu.sync_copy(f2.at[wid], i2)
    pltpu.sync_copy(f3.at[wid], i3)
    pltpu.sync_copy(q0.at[wid], qb)

    zero = jnp.zeros((16,), jnp.float32)

    def zrow(r, carry):
        zbuf[r, pl.ds(0, 16)] = zero
        zbuf[r, pl.ds(16, 16)] = zero
        zbuf[r, pl.ds(32, 16)] = zero
        return carry

    lax.fori_loop(0, IDX_LANES, zrow, 0)

    tabs = (e1, e2, e3)
    idxs = (i1, i2, i3)
    rbufs = (r1, r2, r3)
    iota16 = lax.iota(jnp.int32, 16)

    def e0_copy(c, j):
        half = j % 2
        return pltpu.make_async_copy(
            p0.at[i0.at[c * CHUNK_IDX_ROWS + j]],
            g0.at[pl.ds(half * IDX_LANES, IDX_LANES)],
            sga if half == 0 else sgb,
        )

    def chunk(c, carry):
        cps = []
        for t in range(3):
            for j in range(CHUNK_IDX_ROWS):
                cps.append(pltpu.make_async_copy(
                    tabs[t].at[idxs[t].at[c * CHUNK_IDX_ROWS + j]],
                    rbufs[t].at[pl.ds(j * IDX_LANES, IDX_LANES)],
                    sem,
                ))
        for cp in cps:
            cp.start()

        e0_copy(c, 0).start()
        for j in range(CHUNK_IDX_ROWS):
            if j + 1 < CHUNK_IDX_ROWS:
                e0_copy(c, j + 1).start()
            e0_copy(c, j).wait()
            half_off = (j % 2) * IDX_LANES

            def grp(g, carry2):
                qv = qb[c * CHUNK_IDX_ROWS + j, pl.ds(g * 16, 16)]
                col0 = qv * 32
                rowv = g * 16 + iota16 + half_off
                srow = j * IDX_LANES + g * 16 + iota16
                for k in range(16):
                    for h in (0, 16):
                        cc = ((iota16 + k) & 15) + h
                        v = plsc.load_gather(g0, [rowv, col0 + cc])
                        plsc.store_scatter(r0, [srow, cc], v)
                return carry2

            lax.fori_loop(0, IDX_LANES // 16, grp, 0)

        for cp in cps:
            cp.wait()
        rows = pl.ds(base_r + c * CHUNK, CHUNK)
        pltpu.sync_copy(r0, out.at[rows, pl.ds(0, 32)])
        for t in range(3):
            pltpu.sync_copy(rbufs[t],
                            out.at[rows, pl.ds(OFFS[t + 1], DIMS[t + 1])])
        for j in range(CHUNK_IDX_ROWS):
            pltpu.sync_copy(
                zbuf,
                out.at[pl.ds(base_r + c * CHUNK + j * IDX_LANES, IDX_LANES),
                       pl.ds(80, 48)])
        return carry

    lax.fori_loop(0, NCHUNK, chunk, 0)


@jax.jit
def _sc_gather(g0i, f1, f2, f3, q0, p0, e1, e2, e3):
    mesh = plsc.VectorSubcoreMesh(core_axis_name="c", subcore_axis_name="s")
    return pl.kernel(
        _gather_body,
        out_type=jax.ShapeDtypeStruct((N, PAD), jnp.float32),
        mesh=mesh,
        scratch_types=[
            pltpu.VMEM((IDX_ROWS_PER_W, IDX_LANES), jnp.int32),
            pltpu.VMEM((IDX_ROWS_PER_W, IDX_LANES), jnp.int32),
            pltpu.VMEM((IDX_ROWS_PER_W, IDX_LANES), jnp.int32),
            pltpu.VMEM((IDX_ROWS_PER_W, IDX_LANES), jnp.int32),
            pltpu.VMEM((IDX_ROWS_PER_W, IDX_LANES), jnp.int32),
            pltpu.VMEM((2 * IDX_LANES, PAD), jnp.float32),
            pltpu.VMEM((CHUNK, 32), jnp.float32),
            pltpu.VMEM((CHUNK, 16), jnp.float32),
            pltpu.VMEM((CHUNK, 16), jnp.float32),
            pltpu.VMEM((CHUNK, 16), jnp.float32),
            pltpu.VMEM((IDX_LANES, 48), jnp.float32),
            pltpu.SemaphoreType.DMA,
            pltpu.SemaphoreType.DMA,
            pltpu.SemaphoreType.DMA,
        ],
        compiler_params=pltpu.CompilerParams(
            use_tc_tiling_on_sc=False, needs_layout_passes=False),
    )(g0i, f1, f2, f3, q0, p0, e1, e2, e3)


MM_BLK = 2048


def _mm_body(s, w, bias, o):
    o[...] = jnp.dot(s[...], w[...],
                     preferred_element_type=jnp.float32) + bias[0:1, :]


@jax.jit
def _tc_project(s, w, bias):
    return pl.pallas_call(
        _mm_body,
        grid=(N // MM_BLK,),
        in_specs=[
            pl.BlockSpec((MM_BLK, PAD), lambda i: (i, 0)),
            pl.BlockSpec((PAD, OUT_DIM), lambda i: (0, 0)),
            pl.BlockSpec((8, OUT_DIM), lambda i: (0, 0)),
        ],
        out_specs=pl.BlockSpec((MM_BLK, OUT_DIM), lambda i: (i, 0)),
        out_shape=jax.ShapeDtypeStruct((N, OUT_DIM), jnp.float32),
    )(s, w, bias)


def kernel(feat0, feat1, feat2, feat3, E0, E1, E2, E3, W, b):
    shaped = lambda f: f.reshape(NW, IDX_ROWS_PER_W, IDX_LANES)
    g0i = shaped((feat0 >> PSHIFT) * PROWS + (feat0 & (PROWS - 1)))
    q0 = shaped((feat0 >> QSHIFT) & 3)
    fs = [shaped(f) for f in (feat1, feat2, feat3)]
    P0 = _tc_pack(jnp.transpose(E0))
    s = _sc_gather(g0i, *fs, q0, P0, E1, E2, E3)
    wp = jnp.zeros((PAD, OUT_DIM), jnp.float32).at[0:80, :].set(W)
    bias = jnp.broadcast_to(b, (8, OUT_DIM))
    out = _tc_project(s, wp, bias)
    return out.reshape(B, T, OUT_DIM)


# speedup vs baseline: 6.4058x; 1.0337x over previous
import functools

import jax
import jax.numpy as jnp
from jax import lax
from jax.experimental import pallas as pl
from jax.experimental.pallas import tpu as pltpu
from jax.experimental.pallas import tpu_sc as plsc

B, T = 1024, 200
N = B * T
DIMS = (32, 16, 16, 16)
OFFS = (0, 32, 48, 64)
PAD = 128
OUT_DIM = 128

NC, NS = 2, 16
NW = NC * NS
ROWS_PER_W = N // NW
IDX_LANES = 128
IDX_ROWS_PER_W = ROWS_PER_W // IDX_LANES
CHUNK_IDX_ROWS = 5
CHUNK = CHUNK_IDX_ROWS * IDX_LANES
NCHUNK = IDX_ROWS_PER_W // CHUNK_IDX_ROWS

V0 = 1000000
PBN = 8192
PROWS = PBN // 4
NPACK = (V0 + PBN - 1) // PBN
P0_ROWS = NPACK * PROWS
PSHIFT = 13
QSHIFT = 11


def _pack_body(x, o):
    acc = None
    for g in range(4):
        y = lax.dot_general(
            x[:, PROWS * g:PROWS * (g + 1)],
            jnp.eye(32, 128, 32 * g, dtype=jnp.float32),
            (((0,), (0,)), ((), ())),
            preferred_element_type=jnp.float32)
        acc = y if acc is None else acc + y
    o[...] = acc


@jax.jit
def _tc_pack(e0t):
    return pl.pallas_call(
        _pack_body,
        grid=(NPACK,),
        in_specs=[pl.BlockSpec((32, PBN), lambda i: (0, i))],
        out_specs=pl.BlockSpec((PROWS, PAD), lambda i: (i, 0)),
        out_shape=jax.ShapeDtypeStruct((P0_ROWS, PAD), jnp.float32),
    )(e0t)


def _gather_body(g0i, f1, f2, f3, q0, p0, e1, e2, e3, out,
                 i0, i1, i2, i3, qb, g0, r0, r1, r2, r3,
                 sem, sga, sgb):
    wid = lax.axis_index("s") * NC + lax.axis_index("c")
    base_r = wid * ROWS_PER_W

    pltpu.sync_copy(g0i.at[wid], i0)
    pltpu.sync_copy(f1.at[wid], i1)
    pltpu.sync_copy(f2.at[wid], i2)
    pltpu.sync_copy(f3.at[wid], i3)
    pltpu.sync_copy(q0.at[wid], qb)

    tabs = (e1, e2, e3)
    idxs = (i1, i2, i3)
    rbufs = (r1, r2, r3)
    iota16 = lax.iota(jnp.int32, 16)

    def e0_copy(c, j):
        half = j % 2
        return pltpu.make_async_copy(
            p0.at[i0.at[c * CHUNK_IDX_ROWS + j]],
            g0.at[pl.ds(half * IDX_LANES, IDX_LANES)],
            sga if half == 0 else sgb,
        )

    def chunk(c, carry):
        cps = []
        for t in range(3):
            for j in range(CHUNK_IDX_ROWS):
                cps.append(pltpu.make_async_copy(
                    tabs[t].at[idxs[t].at[c * CHUNK_IDX_ROWS + j]],
                    rbufs[t].at[pl.ds(j * IDX_LANES, IDX_LANES)],
                    sem,
                ))
        for cp in cps:
            cp.start()

        e0_copy(c, 0).start()
        for j in range(CHUNK_IDX_ROWS):
            if j + 1 < CHUNK_IDX_ROWS:
                e0_copy(c, j + 1).start()
            e0_copy(c, j).wait()
            half_off = (j % 2) * IDX_LANES

            def grp(g, carry2):
                qv = qb[c * CHUNK_IDX_ROWS + j, pl.ds(g * 16, 16)]
                col0 = qv * 32
                rowv = g * 16 + iota16 + half_off
                srow = j * IDX_LANES + g * 16 + iota16
                for k in range(16):
                    for h in (0, 16):
                        cc = ((iota16 + k) & 15) + h
                        v = plsc.load_gather(g0, [rowv, col0 + cc])
                        plsc.store_scatter(r0, [srow, cc], v)
                return carry2

            lax.fori_loop(0, IDX_LANES // 16, grp, 0)

        for cp in cps:
            cp.wait()
        rows = pl.ds(base_r + c * CHUNK, CHUNK)
        pltpu.sync_copy(r0, out.at[rows, pl.ds(0, 32)])
        for t in range(3):
            pltpu.sync_copy(rbufs[t],
                            out.at[rows, pl.ds(OFFS[t + 1], DIMS[t + 1])])
        return carry

    lax.fori_loop(0, NCHUNK, chunk, 0)


@jax.jit
def _sc_gather(g0i, f1, f2, f3, q0, p0, e1, e2, e3):
    mesh = plsc.VectorSubcoreMesh(core_axis_name="c", subcore_axis_name="s")
    return pl.kernel(
        _gather_body,
        out_type=jax.ShapeDtypeStruct((N, PAD), jnp.float32),
        mesh=mesh,
        scratch_types=[
            pltpu.VMEM((IDX_ROWS_PER_W, IDX_LANES), jnp.int32),
            pltpu.VMEM((IDX_ROWS_PER_W, IDX_LANES), jnp.int32),
            pltpu.VMEM((IDX_ROWS_PER_W, IDX_LANES), jnp.int32),
            pltpu.VMEM((IDX_ROWS_PER_W, IDX_LANES), jnp.int32),
            pltpu.VMEM((IDX_ROWS_PER_W, IDX_LANES), jnp.int32),
            pltpu.VMEM((2 * IDX_LANES, PAD), jnp.float32),
            pltpu.VMEM((CHUNK, 32), jnp.float32),
            pltpu.VMEM((CHUNK, 16), jnp.float32),
            pltpu.VMEM((CHUNK, 16), jnp.float32),
            pltpu.VMEM((CHUNK, 16), jnp.float32),
            pltpu.SemaphoreType.DMA,
            pltpu.SemaphoreType.DMA,
            pltpu.SemaphoreType.DMA,
        ],
        compiler_params=pltpu.CompilerParams(
            use_tc_tiling_on_sc=False, needs_layout_passes=False),
    )(g0i, f1, f2, f3, q0, p0, e1, e2, e3)


MM_BLK = 2048


def _mm_body(s, w, bias, o):
    lane = lax.broadcasted_iota(jnp.int32, (MM_BLK, PAD), 1)
    sv = jnp.where(lane < 80, s[...], 0.0)
    o[...] = jnp.dot(sv, w[...],
                     preferred_element_type=jnp.float32) + bias[0:1, :]


@jax.jit
def _tc_project(s, w, bias):
    return pl.pallas_call(
        _mm_body,
        grid=(N // MM_BLK,),
        in_specs=[
            pl.BlockSpec((MM_BLK, PAD), lambda i: (i, 0)),
            pl.BlockSpec((PAD, OUT_DIM), lambda i: (0, 0)),
            pl.BlockSpec((8, OUT_DIM), lambda i: (0, 0)),
        ],
        out_specs=pl.BlockSpec((MM_BLK, OUT_DIM), lambda i: (i, 0)),
        out_shape=jax.ShapeDtypeStruct((N, OUT_DIM), jnp.float32),
    )(s, w, bias)


def kernel(feat0, feat1, feat2, feat3, E0, E1, E2, E3, W, b):
    shaped = lambda f: f.reshape(NW, IDX_ROWS_PER_W, IDX_LANES)
    g0i = shaped((feat0 >> PSHIFT) * PROWS + (feat0 & (PROWS - 1)))
    q0 = shaped((feat0 >> QSHIFT) & 3)
    fs = [shaped(f) for f in (feat1, feat2, feat3)]
    P0 = _tc_pack(jnp.transpose(E0))
    s = _sc_gather(g0i, *fs, q0, P0, E1, E2, E3)
    wp = jnp.zeros((PAD, OUT_DIM), jnp.float32).at[0:80, :].set(W)
    bias = jnp.broadcast_to(b, (8, OUT_DIM))
    out = _tc_project(s, wp, bias)
    return out.reshape(B, T, OUT_DIM)


# speedup vs baseline: 7.3725x; 1.1509x over previous
import functools

import jax
import jax.numpy as jnp
from jax import lax
from jax.experimental import pallas as pl
from jax.experimental.pallas import tpu as pltpu
from jax.experimental.pallas import tpu_sc as plsc

B, T = 1024, 200
N = B * T
DIMS = (32, 16, 16, 16)
OFFS = (0, 32, 48, 64)
PAD = 128
OUT_DIM = 128

NC, NS = 2, 16
NW = NC * NS
ROWS_PER_W = N // NW
IDX_LANES = 128
IDX_ROWS_PER_W = ROWS_PER_W // IDX_LANES
CHUNK_IDX_ROWS = 5
CHUNK = CHUNK_IDX_ROWS * IDX_LANES
NCHUNK = IDX_ROWS_PER_W // CHUNK_IDX_ROWS

V0 = 1000000
PBN = 16384
PROWS = PBN // 4
NPACK = (V0 + PBN - 1) // PBN
P0_ROWS = NPACK * PROWS
PSHIFT = 14
QSHIFT = 12


def _pack_body(x, o):
    acc = None
    for g in range(4):
        y = lax.dot_general(
            x[:, PROWS * g:PROWS * (g + 1)],
            jnp.eye(32, 128, 32 * g, dtype=jnp.float32),
            (((0,), (0,)), ((), ())),
            preferred_element_type=jnp.float32)
        acc = y if acc is None else acc + y
    o[...] = acc


@jax.jit
def _tc_pack(e0t):
    return pl.pallas_call(
        _pack_body,
        grid=(NPACK,),
        in_specs=[pl.BlockSpec((32, PBN), lambda i: (0, i))],
        out_specs=pl.BlockSpec((PROWS, PAD), lambda i: (i, 0)),
        out_shape=jax.ShapeDtypeStruct((P0_ROWS, PAD), jnp.float32),
    )(e0t)


def _gather_body(g0i, f1, f2, f3, q0, p0, e1, e2, e3, out,
                 i0, i1, i2, i3, qb, g0, r0, r1, r2, r3,
                 sem, sga, sgb):
    wid = lax.axis_index("s") * NC + lax.axis_index("c")
    base_r = wid * ROWS_PER_W

    pltpu.sync_copy(g0i.at[wid], i0)
    pltpu.sync_copy(f1.at[wid], i1)
    pltpu.sync_copy(f2.at[wid], i2)
    pltpu.sync_copy(f3.at[wid], i3)
    pltpu.sync_copy(q0.at[wid], qb)

    tabs = (e1, e2, e3)
    idxs = (i1, i2, i3)
    rbufs = (r1, r2, r3)
    iota16 = lax.iota(jnp.int32, 16)

    def e0_copy(c, j):
        half = j % 2
        return pltpu.make_async_copy(
            p0.at[i0.at[c * CHUNK_IDX_ROWS + j]],
            g0.at[pl.ds(half * IDX_LANES, IDX_LANES)],
            sga if half == 0 else sgb,
        )

    def chunk(c, carry):
        cps = []
        for t in range(3):
            for j in range(CHUNK_IDX_ROWS):
                cps.append(pltpu.make_async_copy(
                    tabs[t].at[idxs[t].at[c * CHUNK_IDX_ROWS + j]],
                    rbufs[t].at[pl.ds(j * IDX_LANES, IDX_LANES)],
                    sem,
                ))
        for cp in cps:
            cp.start()

        e0_copy(c, 0).start()
        for j in range(CHUNK_IDX_ROWS):
            if j + 1 < CHUNK_IDX_ROWS:
                e0_copy(c, j + 1).start()
            e0_copy(c, j).wait()
            half_off = (j % 2) * IDX_LANES

            def grp(g, carry2):
                qv = qb[c * CHUNK_IDX_ROWS + j, pl.ds(g * 16, 16)]
                col0 = qv * 32
                rowv = g * 16 + iota16 + half_off
                srow = j * IDX_LANES + g * 16 + iota16
                for k in range(16):
                    for h in (0, 16):
                        cc = ((iota16 + k) & 15) + h
                        v = plsc.load_gather(g0, [rowv, col0 + cc])
                        plsc.store_scatter(r0, [srow, cc], v)
                return carry2

            lax.fori_loop(0, IDX_LANES // 16, grp, 0)

        for cp in cps:
            cp.wait()
        rows = pl.ds(base_r + c * CHUNK, CHUNK)
        pltpu.sync_copy(r0, out.at[rows, pl.ds(0, 32)])
        for t in range(3):
            pltpu.sync_copy(rbufs[t],
                            out.at[rows, pl.ds(OFFS[t + 1], DIMS[t + 1])])
        return carry

    lax.fori_loop(0, NCHUNK, chunk, 0)


@jax.jit
def _sc_gather(g0i, f1, f2, f3, q0, p0, e1, e2, e3):
    mesh = plsc.VectorSubcoreMesh(core_axis_name="c", subcore_axis_name="s")
    return pl.kernel(
        _gather_body,
        out_type=jax.ShapeDtypeStruct((N, PAD), jnp.float32),
        mesh=mesh,
        scratch_types=[
            pltpu.VMEM((IDX_ROWS_PER_W, IDX_LANES), jnp.int32),
            pltpu.VMEM((IDX_ROWS_PER_W, IDX_LANES), jnp.int32),
            pltpu.VMEM((IDX_ROWS_PER_W, IDX_LANES), jnp.int32),
            pltpu.VMEM((IDX_ROWS_PER_W, IDX_LANES), jnp.int32),
            pltpu.VMEM((IDX_ROWS_PER_W, IDX_LANES), jnp.int32),
            pltpu.VMEM((2 * IDX_LANES, PAD), jnp.float32),
            pltpu.VMEM((CHUNK, 32), jnp.float32),
            pltpu.VMEM((CHUNK, 16), jnp.float32),
            pltpu.VMEM((CHUNK, 16), jnp.float32),
            pltpu.VMEM((CHUNK, 16), jnp.float32),
            pltpu.SemaphoreType.DMA,
            pltpu.SemaphoreType.DMA,
            pltpu.SemaphoreType.DMA,
        ],
        compiler_params=pltpu.CompilerParams(
            use_tc_tiling_on_sc=False, needs_layout_passes=False),
    )(g0i, f1, f2, f3, q0, p0, e1, e2, e3)


MM_BLK = 4096


def _mm_body(s, w, bias, o):
    lane = lax.broadcasted_iota(jnp.int32, (MM_BLK, PAD), 1)
    sv = jnp.where(lane < 80, s[...], 0.0)
    o[...] = jnp.dot(sv, w[...],
                     preferred_element_type=jnp.float32) + bias[0:1, :]


@jax.jit
def _tc_project(s, w, bias):
    return pl.pallas_call(
        _mm_body,
        grid=(N // MM_BLK,),
        in_specs=[
            pl.BlockSpec((MM_BLK, PAD), lambda i: (i, 0)),
            pl.BlockSpec((PAD, OUT_DIM), lambda i: (0, 0)),
            pl.BlockSpec((8, OUT_DIM), lambda i: (0, 0)),
        ],
        out_specs=pl.BlockSpec((MM_BLK, OUT_DIM), lambda i: (i, 0)),
        out_shape=jax.ShapeDtypeStruct((N, OUT_DIM), jnp.float32),
    )(s, w, bias)


def kernel(feat0, feat1, feat2, feat3, E0, E1, E2, E3, W, b):
    shaped = lambda f: f.reshape(NW, IDX_ROWS_PER_W, IDX_LANES)
    g0i = shaped((feat0 >> PSHIFT) * PROWS + (feat0 & (PROWS - 1)))
    q0 = shaped((feat0 >> QSHIFT) & 3)
    fs = [shaped(f) for f in (feat1, feat2, feat3)]
    P0 = _tc_pack(jnp.transpose(E0))
    s = _sc_gather(g0i, *fs, q0, P0, E1, E2, E3)
    wp = jnp.zeros((PAD, OUT_DIM), jnp.float32).at[0:80, :].set(W)
    bias = jnp.broadcast_to(b, (8, OUT_DIM))
    out = _tc_project(s, wp, bias)
    return out.reshape(B, T, OUT_DIM)


# speedup vs baseline: 7.6819x; 1.0420x over previous
import functools

import jax
import jax.numpy as jnp
from jax import lax
from jax.experimental import pallas as pl
from jax.experimental.pallas import tpu as pltpu
from jax.experimental.pallas import tpu_sc as plsc

B, T = 1024, 200
N = B * T
DIMS = (32, 16, 16, 16)
OFFS = (0, 32, 48, 64)
PAD = 128
OUT_DIM = 128

NC, NS = 2, 16
NW = NC * NS
ROWS_PER_W = N // NW
IDX_LANES = 128
IDX_ROWS_PER_W = ROWS_PER_W // IDX_LANES
CHUNK_IDX_ROWS = 5
CHUNK = CHUNK_IDX_ROWS * IDX_LANES
NCHUNK = IDX_ROWS_PER_W // CHUNK_IDX_ROWS

V0 = 1000000
PBN = 32768
PROWS = PBN // 4
NPACK = (V0 + PBN - 1) // PBN
P0_ROWS = NPACK * PROWS
PSHIFT = 15
QSHIFT = 13


def _pack_body(x, o):
    acc = None
    for g in range(4):
        y = lax.dot_general(
            x[:, PROWS * g:PROWS * (g + 1)],
            jnp.eye(32, 128, 32 * g, dtype=jnp.float32),
            (((0,), (0,)), ((), ())),
            preferred_element_type=jnp.float32)
        acc = y if acc is None else acc + y
    o[...] = acc


@jax.jit
def _tc_pack(e0t):
    return pl.pallas_call(
        _pack_body,
        grid=(NPACK,),
        in_specs=[pl.BlockSpec((32, PBN), lambda i: (0, i))],
        out_specs=pl.BlockSpec((PROWS, PAD), lambda i: (i, 0)),
        out_shape=jax.ShapeDtypeStruct((P0_ROWS, PAD), jnp.float32),
    )(e0t)


def _gather_body(g0i, f1, f2, f3, q0, p0, e1, e2, e3, out,
                 i0, i1, i2, i3, qb, g0, r0, r1, r2, r3,
                 sem, sga, sgb):
    wid = lax.axis_index("s") * NC + lax.axis_index("c")
    base_r = wid * ROWS_PER_W

    pltpu.sync_copy(g0i.at[wid], i0)
    pltpu.sync_copy(f1.at[wid], i1)
    pltpu.sync_copy(f2.at[wid], i2)
    pltpu.sync_copy(f3.at[wid], i3)
    pltpu.sync_copy(q0.at[wid], qb)

    tabs = (e1, e2, e3)
    idxs = (i1, i2, i3)
    rbufs = (r1, r2, r3)
    iota16 = lax.iota(jnp.int32, 16)

    def e0_copy(c, j):
        half = j % 2
        return pltpu.make_async_copy(
            p0.at[i0.at[c * CHUNK_IDX_ROWS + j]],
            g0.at[pl.ds(half * IDX_LANES, IDX_LANES)],
            sga if half == 0 else sgb,
        )

    def chunk(c, carry):
        cps = []
        for t in range(3):
            for j in range(CHUNK_IDX_ROWS):
                cps.append(pltpu.make_async_copy(
                    tabs[t].at[idxs[t].at[c * CHUNK_IDX_ROWS + j]],
                    rbufs[t].at[pl.ds(j * IDX_LANES, IDX_LANES)],
                    sem,
                ))
        for cp in cps:
            cp.start()

        e0_copy(c, 0).start()
        for j in range(CHUNK_IDX_ROWS):
            if j + 1 < CHUNK_IDX_ROWS:
                e0_copy(c, j + 1).start()
            e0_copy(c, j).wait()
            half_off = (j % 2) * IDX_LANES

            def grp(g, carry2):
                qv = qb[c * CHUNK_IDX_ROWS + j, pl.ds(g * 16, 16)]
                col0 = qv * 32
                rowv = g * 16 + iota16 + half_off
                srow = j * IDX_LANES + g * 16 + iota16
                for k in range(16):
                    for h in (0, 16):
                        cc = ((iota16 + k) & 15) + h
                        v = plsc.load_gather(g0, [rowv, col0 + cc])
                        plsc.store_scatter(r0, [srow, cc], v)
                return carry2

            lax.fori_loop(0, IDX_LANES // 16, grp, 0)

        for cp in cps:
            cp.wait()
        rows = pl.ds(base_r + c * CHUNK, CHUNK)
        pltpu.sync_copy(r0, out.at[rows, pl.ds(0, 32)])
        for t in range(3):
            pltpu.sync_copy(rbufs[t],
                            out.at[rows, pl.ds(OFFS[t + 1], DIMS[t + 1])])
        return carry

    lax.fori_loop(0, NCHUNK, chunk, 0)


@jax.jit
def _sc_gather(g0i, f1, f2, f3, q0, p0, e1, e2, e3):
    mesh = plsc.VectorSubcoreMesh(core_axis_name="c", subcore_axis_name="s")
    return pl.kernel(
        _gather_body,
        out_type=jax.ShapeDtypeStruct((N, PAD), jnp.float32),
        mesh=mesh,
        scratch_types=[
            pltpu.VMEM((IDX_ROWS_PER_W, IDX_LANES), jnp.int32),
            pltpu.VMEM((IDX_ROWS_PER_W, IDX_LANES), jnp.int32),
            pltpu.VMEM((IDX_ROWS_PER_W, IDX_LANES), jnp.int32),
            pltpu.VMEM((IDX_ROWS_PER_W, IDX_LANES), jnp.int32),
            pltpu.VMEM((IDX_ROWS_PER_W, IDX_LANES), jnp.int32),
            pltpu.VMEM((2 * IDX_LANES, PAD), jnp.float32),
            pltpu.VMEM((CHUNK, 32), jnp.float32),
            pltpu.VMEM((CHUNK, 16), jnp.float32),
            pltpu.VMEM((CHUNK, 16), jnp.float32),
            pltpu.VMEM((CHUNK, 16), jnp.float32),
            pltpu.SemaphoreType.DMA,
            pltpu.SemaphoreType.DMA,
            pltpu.SemaphoreType.DMA,
        ],
        compiler_params=pltpu.CompilerParams(
            use_tc_tiling_on_sc=False, needs_layout_passes=False),
    )(g0i, f1, f2, f3, q0, p0, e1, e2, e3)


MM_BLK = 8192


def _mm_body(s, w, bias, o):
    lane = lax.broadcasted_iota(jnp.int32, (MM_BLK, PAD), 1)
    sv = jnp.where(lane < 80, s[...], 0.0)
    o[...] = jnp.dot(sv, w[...],
                     preferred_element_type=jnp.float32) + bias[0:1, :]


@jax.jit
def _tc_project(s, w, bias):
    return pl.pallas_call(
        _mm_body,
        grid=(N // MM_BLK,),
        in_specs=[
            pl.BlockSpec((MM_BLK, PAD), lambda i: (i, 0)),
            pl.BlockSpec((PAD, OUT_DIM), lambda i: (0, 0)),
            pl.BlockSpec((8, OUT_DIM), lambda i: (0, 0)),
        ],
        out_specs=pl.BlockSpec((MM_BLK, OUT_DIM), lambda i: (i, 0)),
        out_shape=jax.ShapeDtypeStruct((N, OUT_DIM), jnp.float32),
    )(s, w, bias)


def kernel(feat0, feat1, feat2, feat3, E0, E1, E2, E3, W, b):
    shaped = lambda f: f.reshape(NW, IDX_ROWS_PER_W, IDX_LANES)
    g0i = shaped((feat0 >> PSHIFT) * PROWS + (feat0 & (PROWS - 1)))
    q0 = shaped((feat0 >> QSHIFT) & 3)
    fs = [shaped(f) for f in (feat1, feat2, feat3)]
    P0 = _tc_pack(jnp.transpose(E0))
    s = _sc_gather(g0i, *fs, q0, P0, E1, E2, E3)
    wp = jnp.zeros((PAD, OUT_DIM), jnp.float32).at[0:80, :].set(W)
    bias = jnp.broadcast_to(b, (8, OUT_DIM))
    out = _tc_project(s, wp, bias)
    return out.reshape(B, T, OUT_DIM)


# speedup vs baseline: 9.3132x; 1.2124x over previous
import functools

import jax
import jax.numpy as jnp
from jax import lax
from jax.experimental import pallas as pl
from jax.experimental.pallas import tpu as pltpu
from jax.experimental.pallas import tpu_sc as plsc

B, T = 1024, 200
N = B * T
DIMS = (32, 16, 16, 16)
OFFS = (0, 32, 48, 64)
PAD = 128
OUT_DIM = 128

NC, NS = 2, 16
NW = NC * NS
ROWS_PER_W = N // NW
IDX_LANES = 128
IDX_ROWS_PER_W = ROWS_PER_W // IDX_LANES
CHUNK_IDX_ROWS = 5
CHUNK = CHUNK_IDX_ROWS * IDX_LANES
NCHUNK = IDX_ROWS_PER_W // CHUNK_IDX_ROWS

PBN0 = 32768
PBN1 = 8192


def _make_pack(d, pbn, v):
    r = 128 // d
    prows = pbn // r
    nb = -(-v // pbn)

    def body(x, o):
        acc = None
        for g in range(r):
            y = lax.dot_general(
                x[:, prows * g:prows * (g + 1)],
                jnp.eye(d, 128, d * g, dtype=jnp.float32),
                (((0,), (0,)), ((), ())),
                preferred_element_type=jnp.float32)
            acc = y if acc is None else acc + y
        o[...] = acc

    @jax.jit
    def pack(et):
        return pl.pallas_call(
            body,
            grid=(nb,),
            in_specs=[pl.BlockSpec((d, pbn), lambda i: (0, i))],
            out_specs=pl.BlockSpec((prows, PAD), lambda i: (i, 0)),
            out_shape=jax.ShapeDtypeStruct((nb * prows, PAD), jnp.float32),
        )(et)

    def pidx(t):
        return (prows * (t // pbn) + t % prows) * r + (t % pbn) // prows

    return pack, pidx, nb * prows * r


_pack0, _pidx0, P0V = _make_pack(32, PBN0, 1000000)
_pack1, _pidx1, P1V = _make_pack(16, PBN1, 100000)


def _gather_body(f0, f1, f2, f3, e0, e1, e2, e3, out,
                 i0, i1, i2, i3, r0, r1, r2, r3, sem):
    wid = lax.axis_index("s") * NC + lax.axis_index("c")
    base_r = wid * ROWS_PER_W

    pltpu.sync_copy(f0.at[wid], i0)
    pltpu.sync_copy(f1.at[wid], i1)
    pltpu.sync_copy(f2.at[wid], i2)
    pltpu.sync_copy(f3.at[wid], i3)

    tabs = (e0, e1, e2, e3)
    idxs = (i0, i1, i2, i3)
    rbufs = (r0, r1, r2, r3)

    def chunk(c, carry):
        cps = []
        for t in range(4):
            for j in range(CHUNK_IDX_ROWS):
                cps.append(pltpu.make_async_copy(
                    tabs[t].at[idxs[t].at[c * CHUNK_IDX_ROWS + j]],
                    rbufs[t].at[pl.ds(j * IDX_LANES, IDX_LANES)],
                    sem,
                ))
        for cp in cps:
            cp.start()
        for cp in cps:
            cp.wait()
        rows = pl.ds(base_r + c * CHUNK, CHUNK)
        for t in range(4):
            pltpu.sync_copy(rbufs[t], out.at[rows, pl.ds(OFFS[t], DIMS[t])])
        return carry

    lax.fori_loop(0, NCHUNK, chunk, 0)


@jax.jit
def _sc_gather(f0, f1, f2, f3, e0, e1, e2, e3):
    mesh = plsc.VectorSubcoreMesh(core_axis_name="c", subcore_axis_name="s")
    return pl.kernel(
        _gather_body,
        out_type=jax.ShapeDtypeStruct((N, PAD), jnp.float32),
        mesh=mesh,
        scratch_types=[
            pltpu.VMEM((IDX_ROWS_PER_W, IDX_LANES), jnp.int32),
            pltpu.VMEM((IDX_ROWS_PER_W, IDX_LANES), jnp.int32),
            pltpu.VMEM((IDX_ROWS_PER_W, IDX_LANES), jnp.int32),
            pltpu.VMEM((IDX_ROWS_PER_W, IDX_LANES), jnp.int32),
            pltpu.VMEM((CHUNK, 32), jnp.float32),
            pltpu.VMEM((CHUNK, 16), jnp.float32),
            pltpu.VMEM((CHUNK, 16), jnp.float32),
            pltpu.VMEM((CHUNK, 16), jnp.float32),
            pltpu.SemaphoreType.DMA,
        ],
        compiler_params=pltpu.CompilerParams(
            use_tc_tiling_on_sc=False, needs_layout_passes=False),
    )(f0, f1, f2, f3, e0, e1, e2, e3)


MM_BLK = 8192


def _mm_body(s, w, bias, o):
    lane = lax.broadcasted_iota(jnp.int32, (MM_BLK, PAD), 1)
    sv = jnp.where(lane < 80, s[...], 0.0)
    o[...] = jnp.dot(sv, w[...],
                     preferred_element_type=jnp.float32) + bias[0:1, :]


@jax.jit
def _tc_project(s, w, bias):
    return pl.pallas_call(
        _mm_body,
        grid=(N // MM_BLK,),
        in_specs=[
            pl.BlockSpec((MM_BLK, PAD), lambda i: (i, 0)),
            pl.BlockSpec((PAD, OUT_DIM), lambda i: (0, 0)),
            pl.BlockSpec((8, OUT_DIM), lambda i: (0, 0)),
        ],
        out_specs=pl.BlockSpec((MM_BLK, OUT_DIM), lambda i: (i, 0)),
        out_shape=jax.ShapeDtypeStruct((N, OUT_DIM), jnp.float32),
    )(s, w, bias)


def kernel(feat0, feat1, feat2, feat3, E0, E1, E2, E3, W, b):
    shaped = lambda f: f.reshape(NW, IDX_ROWS_PER_W, IDX_LANES)
    g0 = shaped(_pidx0(feat0))
    g1 = shaped(_pidx1(feat1))
    g2 = shaped(_pidx1(feat2))
    g3 = shaped(feat3)
    P0 = _pack0(jnp.transpose(E0)).reshape(P0V, 32)
    P1 = _pack1(jnp.transpose(E1)).reshape(P1V, 16)
    P2 = _pack1(jnp.transpose(E2)).reshape(P1V, 16)
    s = _sc_gather(g0, g1, g2, g3, P0, P1, P2, E3)
    wp = jnp.zeros((PAD, OUT_DIM), jnp.float32).at[0:80, :].set(W)
    bias = jnp.broadcast_to(b, (8, OUT_DIM))
    out = _tc_project(s, wp, bias)
    return out.reshape(B, T, OUT_DIM)
